# Initial kernel scaffold; baseline (speedup 1.0000x reference)
import functools

import jax
import jax.numpy as jnp
from jax import lax
from jax.experimental import pallas as pl
from jax.experimental.pallas import tpu as pltpu
from jax.experimental.pallas import tpu_sc as plsc

NUM_ROWS = 16384 * 50
DIM = 32

NUM_WORKERS = 32
PER_WORKER = NUM_ROWS // NUM_WORKERS
CHUNK = 1600
NUM_CHUNKS = PER_WORKER // CHUNK


def _make_lookup():
    mesh = plsc.VectorSubcoreMesh(core_axis_name="c", subcore_axis_name="s")

    @functools.partial(
        pl.kernel,
        out_type=jax.ShapeDtypeStruct((NUM_ROWS, DIM), jnp.float32),
        mesh=mesh,
        scratch_types=[
            pltpu.VMEM((CHUNK,), jnp.int32),
            pltpu.VMEM((CHUNK, DIM), jnp.float32),
            pltpu.SemaphoreType.DMA,
        ],
    )
    def lookup(table_hbm, idx_hbm, out_hbm, idx_v, rows_v, sem):
        wid = lax.axis_index("s") * 2 + lax.axis_index("c")
        base = wid * PER_WORKER

        @pl.loop(0, NUM_CHUNKS)
        def _chunk(i):
            off = base + i * CHUNK
            pltpu.sync_copy(idx_hbm.at[pl.ds(off, CHUNK)], idx_v)
            pltpu.async_copy(table_hbm.at[idx_v], rows_v, sem).wait()
            pltpu.sync_copy(rows_v, out_hbm.at[pl.ds(off, CHUNK)])

    return lookup


_lookup = _make_lookup()


def kernel(weight, indices):
    flat_idx = indices.reshape(NUM_ROWS).astype(jnp.int32)
    out = _lookup(weight, flat_idx)
    return out.reshape(indices.shape + (DIM,))


# speedup vs baseline: 1.1025x; 1.1025x over previous
import functools

import jax
import jax.numpy as jnp
from jax import lax
from jax.experimental import pallas as pl
from jax.experimental.pallas import tpu as pltpu
from jax.experimental.pallas import tpu_sc as plsc

NUM_ROWS = 16384 * 50
DIM = 32

NUM_WORKERS = 32
PER_WORKER = NUM_ROWS // NUM_WORKERS
CHUNK = 1600
NUM_CHUNKS = PER_WORKER // CHUNK


def _make_lookup():
    mesh = plsc.VectorSubcoreMesh(core_axis_name="c", subcore_axis_name="s")

    @functools.partial(
        pl.kernel,
        out_type=jax.ShapeDtypeStruct((NUM_ROWS, DIM), jnp.float32),
        mesh=mesh,
        scratch_types=[
            pltpu.VMEM((CHUNK,), jnp.int32),
            pltpu.VMEM((CHUNK, DIM), jnp.float32),
            pltpu.SemaphoreType.DMA,
        ],
        compiler_params=pltpu.CompilerParams(use_tc_tiling_on_sc=False),
    )
    def lookup(table_hbm, idx_hbm, out_hbm, idx_v, rows_v, sem):
        wid = lax.axis_index("s") * 2 + lax.axis_index("c")
        base = wid * PER_WORKER

        @pl.loop(0, NUM_CHUNKS)
        def _chunk(i):
            off = base + i * CHUNK
            pltpu.sync_copy(idx_hbm.at[pl.ds(off, CHUNK)], idx_v)
            pltpu.async_copy(table_hbm.at[idx_v], rows_v, sem).wait()
            pltpu.sync_copy(rows_v, out_hbm.at[pl.ds(off, CHUNK)])

    return lookup


_lookup = _make_lookup()


def kernel(weight, indices):
    flat_idx = indices.reshape(NUM_ROWS).astype(jnp.int32)
    out = _lookup(weight, flat_idx)
    return out.reshape(indices.shape + (DIM,))


# speedup vs baseline: 1.1139x; 1.0104x over previous
import functools

import jax
import jax.numpy as jnp
from jax import lax
from jax.experimental import pallas as pl
from jax.experimental.pallas import tpu as pltpu
from jax.experimental.pallas import tpu_sc as plsc

NUM_ROWS = 16384 * 50
DIM = 32

NUM_WORKERS = 32
PER_WORKER = NUM_ROWS // NUM_WORKERS
CHUNK = 800
NBUF = 4
NUM_CHUNKS = PER_WORKER // CHUNK


def _make_lookup():
    mesh = plsc.VectorSubcoreMesh(core_axis_name="c", subcore_axis_name="s")

    @functools.partial(
        pl.kernel,
        out_type=jax.ShapeDtypeStruct((NUM_ROWS, DIM), jnp.float32),
        mesh=mesh,
        scratch_types=[
            pltpu.VMEM((PER_WORKER,), jnp.int32),
            [pltpu.VMEM((CHUNK, DIM), jnp.float32) for _ in range(NBUF)],
            [pltpu.SemaphoreType.DMA for _ in range(NBUF)],
            [pltpu.SemaphoreType.DMA for _ in range(NBUF)],
        ],
        compiler_params=pltpu.CompilerParams(use_tc_tiling_on_sc=False),
    )
    def lookup(table_hbm, idx_hbm, out_hbm, idx_all, rows, gsem, ssem):
        wid = lax.axis_index("s") * 2 + lax.axis_index("c")
        base = wid * PER_WORKER
        pltpu.sync_copy(idx_hbm.at[pl.ds(base, PER_WORKER)], idx_all)

        def start_gather(i, b):
            pltpu.async_copy(
                table_hbm.at[idx_all.at[pl.ds(i * CHUNK, CHUNK)]],
                rows[b],
                gsem[b],
            )

        def start_store(i, b):
            pltpu.async_copy(
                rows[b], out_hbm.at[pl.ds(base + i * CHUNK, CHUNK)], ssem[b]
            )

        for b in range(NBUF):
            start_gather(b, b)

        @pl.loop(0, NUM_CHUNKS // NBUF - 1)
        def _ring(g):
            for b in range(NBUF):
                i = g * NBUF + b
                pltpu.make_async_copy(
                    table_hbm.at[idx_all.at[pl.ds(0, CHUNK)]], rows[b], gsem[b]
                ).wait()
                start_store(i, b)
                pltpu.make_async_copy(
                    rows[b], out_hbm.at[pl.ds(base, CHUNK)], ssem[b]
                ).wait()
                start_gather(i + NBUF, b)

        for b in range(NBUF):
            i = NUM_CHUNKS - NBUF + b
            pltpu.make_async_copy(
                table_hbm.at[idx_all.at[pl.ds(0, CHUNK)]], rows[b], gsem[b]
            ).wait()
            start_store(i, b)
        for b in range(NBUF):
            pltpu.make_async_copy(
                rows[b], out_hbm.at[pl.ds(base, CHUNK)], ssem[b]
            ).wait()

    return lookup


_lookup = _make_lookup()


def kernel(weight, indices):
    flat_idx = indices.reshape(NUM_ROWS).astype(jnp.int32)
    out = _lookup(weight, flat_idx)
    return out.reshape(indices.shape + (DIM,))


# speedup vs baseline: 1.7578x; 1.5780x over previous
import functools

import jax
import jax.numpy as jnp
from jax import lax
from jax.experimental import pallas as pl
from jax.experimental.pallas import tpu as pltpu
from jax.experimental.pallas import tpu_sc as plsc

NUM_EMB = 1000000
DIM = 32
BATCH = 16384
SEQ = 50
SEQ_PAD = 64

NUM_WORKERS = 32
IBLK = BATCH // NUM_WORKERS
NBUF = 8


def _make_lookup():
    mesh = plsc.VectorSubcoreMesh(core_axis_name="c", subcore_axis_name="s")

    @functools.partial(
        pl.kernel,
        out_type=jax.ShapeDtypeStruct((BATCH, SEQ, DIM), jnp.float32),
        mesh=mesh,
        scratch_types=[
            pltpu.VMEM((SEQ, IBLK), jnp.int32),
            pltpu.VMEM((IBLK * SEQ_PAD,), jnp.int32),
            [pltpu.VMEM((SEQ, DIM), jnp.float32) for _ in range(NBUF)],
            [pltpu.SemaphoreType.DMA for _ in range(NBUF)],
            [pltpu.SemaphoreType.DMA for _ in range(NBUF)],
        ],
        compiler_params=pltpu.CompilerParams(
            use_tc_tiling_on_sc=False, needs_layout_passes=False
        ),
    )
    def lookup(table_hbm, idxt_hbm, out_hbm, idxb, idxf, rows, gsem, ssem):
        wid = lax.axis_index("s") * 2 + lax.axis_index("c")
        i0 = wid * IBLK
        pltpu.sync_copy(idxt_hbm.at[:, pl.ds(i0, IBLK)], idxb)

        lanes = lax.iota(jnp.int32, 16)

        @pl.loop(0, IBLK)
        def _t(i):
            i_vec = lanes * 0 + i
            for k in range(SEQ_PAD // 16):
                j = jnp.minimum(k * 16 + lanes, SEQ - 1)
                idxf[pl.ds(i * SEQ_PAD + k * 16, 16)] = plsc.load_gather(
                    idxb, [j, i_vec]
                )

        def start_gather(i, b):
            pltpu.async_copy(
                table_hbm.at[idxf.at[pl.ds(i * SEQ_PAD, SEQ)]], rows[b], gsem[b]
            )

        def start_store(i, b):
            pltpu.async_copy(rows[b], out_hbm.at[i0 + i, :, :], ssem[b])

        def wait_gather(b):
            pltpu.make_async_copy(
                table_hbm.at[pl.ds(0, SEQ), :], rows[b], gsem[b]
            ).wait()

        def wait_store(b):
            pltpu.make_async_copy(
                table_hbm.at[pl.ds(0, SEQ), :], rows[b], ssem[b]
            ).wait()

        for b in range(NBUF):
            start_gather(b, b)

        @pl.loop(0, IBLK // NBUF - 1)
        def _ring(g):
            for b in range(NBUF):
                i = g * NBUF + b
                wait_gather(b)
                start_store(i, b)
                wait_store(b)
                start_gather(i + NBUF, b)

        for b in range(NBUF):
            wait_gather(b)
            start_store(IBLK - NBUF + b, b)
        for b in range(NBUF):
            wait_store(b)

    return lookup


_lookup = _make_lookup()


def kernel(weight, indices):
    return _lookup(weight, indices.T.astype(jnp.int32))


# speedup vs baseline: 1.7922x; 1.0196x over previous
import functools

import jax
import jax.numpy as jnp
from jax import lax
from jax.experimental import pallas as pl
from jax.experimental.pallas import tpu as pltpu
from jax.experimental.pallas import tpu_sc as plsc

NUM_EMB = 1000000
DIM = 32
BATCH = 16384
SEQ = 50

NUM_WORKERS = 32
IBLK = BATCH // NUM_WORKERS
NBUF = 8


def _make_lookup():
    mesh = plsc.VectorSubcoreMesh(core_axis_name="c", subcore_axis_name="s")

    @functools.partial(
        pl.kernel,
        out_type=jax.ShapeDtypeStruct((BATCH, SEQ, DIM), jnp.float32),
        mesh=mesh,
        scratch_types=[
            pltpu.VMEM((IBLK, SEQ), jnp.int32),
            [pltpu.VMEM((SEQ, DIM), jnp.float32) for _ in range(NBUF)],
            [pltpu.SemaphoreType.DMA for _ in range(NBUF)],
            [pltpu.SemaphoreType.DMA for _ in range(NBUF)],
        ],
        compiler_params=pltpu.CompilerParams(
            use_tc_tiling_on_sc=False, needs_layout_passes=False
        ),
    )
    def lookup(table_hbm, idx_hbm, out_hbm, idxb, rows, gsem, ssem):
        wid = lax.axis_index("s") * 2 + lax.axis_index("c")
        i0 = wid * IBLK
        pltpu.sync_copy(idx_hbm.at[pl.ds(i0, IBLK), :], idxb)

        def start_gather(i, b):
            pltpu.async_copy(
                table_hbm.at[idxb.at[i, pl.ds(0, SEQ)]], rows[b], gsem[b]
            )

        def start_store(i, b):
            pltpu.async_copy(rows[b], out_hbm.at[i0 + i, :, :], ssem[b])

        def wait_gather(b):
            pltpu.make_async_copy(
                table_hbm.at[pl.ds(0, SEQ), :], rows[b], gsem[b]
            ).wait()

        def wait_store(b):
            pltpu.make_async_copy(
                table_hbm.at[pl.ds(0, SEQ), :], rows[b], ssem[b]
            ).wait()

        for b in range(NBUF):
            start_gather(b, b)

        @pl.loop(0, IBLK // NBUF - 1)
        def _ring(g):
            for b in range(NBUF):
                i = g * NBUF + b
                wait_gather(b)
                start_store(i, b)
                wait_store(b)
                start_gather(i + NBUF, b)

        for b in range(NBUF):
            wait_gather(b)
            start_store(IBLK - NBUF + b, b)
        for b in range(NBUF):
            wait_store(b)

    return lookup


_lookup = _make_lookup()


def kernel(weight, indices):
    return _lookup(weight, indices.astype(jnp.int32))
